# Initial kernel scaffold; baseline (speedup 1.0000x reference)
import jax
import jax.numpy as jnp
from jax.experimental import pallas as pl


def kernel(input_nodes, edge_index, output_nodes, sentence_emb, soft_labels, hard_labels, emb_table, W1, W2, Wc, bc):
    raise NotImplementedError("write your pallas kernel here")


# speedup vs baseline: 8.1201x; 8.1201x over previous
import functools

import jax
import jax.numpy as jnp
from jax import lax
from jax.experimental import pallas as pl
from jax.experimental.pallas import tpu as pltpu
from jax.experimental.pallas import tpu_sc as plsc

N_NODES = 10000
N_EDGES = 320000
NUM_EMB = 100000
D = 128
BERT_D = 384
B = 1024
NCLS = 10
LAM = 0.5

SC_CORES = 2
SC_TILES = 16
NW = SC_CORES * SC_TILES

NP = 10240
RPT = NP // SC_TILES

ECH = 80
EDGE_EPT = N_EDGES // NW
EDGE_NCH = EDGE_EPT // ECH

GCH = 80
GN_CH = N_NODES // GCH
G_PER_TILE = 4

RB = 1024


def _mesh():
    return plsc.VectorSubcoreMesh(
        core_axis_name="c", subcore_axis_name="s",
        num_cores=SC_CORES, num_subcores=SC_TILES)


@functools.partial(
    pl.kernel,
    out_type=(jax.ShapeDtypeStruct((NP, D), jnp.float32),
              jax.ShapeDtypeStruct((NP,), jnp.float32),
              jax.ShapeDtypeStruct((NP,), jnp.float32)),
    mesh=_mesh(),
    scratch_types=[
        pltpu.VMEM((ECH,), jnp.int32),
        pltpu.VMEM((ECH,), jnp.float32),
        pltpu.VMEM((RPT,), jnp.float32),
        pltpu.VMEM((GCH,), jnp.int32),
        pltpu.VMEM((GCH, D), jnp.float32),
        pltpu.VMEM_SHARED((NP,), jnp.float32),
        pltpu.SemaphoreType.DMA,
    ],
)
def _sc_embed_deg(table, innodes, dst, h0_out, dega_out, degb_out,
                  didx_v, ones_v, dslice_v, gidx_v, rows_v,
                  deg_sh, sem):
    c = lax.axis_index("c")
    s = lax.axis_index("s")
    wid = c * SC_TILES + s

    def _z(i, _):
        dslice_v[pl.ds(i * 16, 16)] = jnp.zeros((16,), jnp.float32)
        return 0
    lax.fori_loop(0, RPT // 16, _z, 0)
    pltpu.sync_copy(dslice_v, deg_sh.at[pl.ds(s * RPT, RPT)])

    def _o(i, _):
        ones_v[pl.ds(i * 16, 16)] = jnp.ones((16,), jnp.float32)
        return 0
    lax.fori_loop(0, ECH // 16, _o, 0)
    plsc.subcore_barrier()

    ebase = c * (N_EDGES // SC_CORES) + s * EDGE_EPT

    def _deg(i, _):
        pltpu.sync_copy(dst.at[pl.ds(ebase + i * ECH, ECH)], didx_v)
        pltpu.sync_copy(ones_v, deg_sh.at[didx_v], add=True)
        return 0
    lax.fori_loop(0, EDGE_NCH, _deg, 0)
    plsc.subcore_barrier()

    pltpu.sync_copy(deg_sh.at[pl.ds(s * RPT, RPT)], dslice_v)

    @pl.when(c == 0)
    def _():
        pltpu.sync_copy(dslice_v, dega_out.at[pl.ds(s * RPT, RPT)])

    @pl.when(c == 1)
    def _():
        pltpu.sync_copy(dslice_v, degb_out.at[pl.ds(s * RPT, RPT)])

    for j in range(G_PER_TILE):
        cid = wid + NW * j

        @pl.when(cid < GN_CH)
        def _():
            n0 = cid * GCH
            pltpu.sync_copy(innodes.at[pl.ds(n0, GCH)], gidx_v)
            pltpu.async_copy(table.at[gidx_v], rows_v, sem).wait()
            pltpu.sync_copy(rows_v, h0_out.at[pl.ds(n0, GCH), :])

    @pl.when(wid < (NP - N_NODES) // GCH)
    def _():
        def _zr(r, _):
            for k in range(D // 16):
                rows_v[r, pl.ds(k * 16, 16)] = jnp.zeros((16,), jnp.float32)
            return 0
        lax.fori_loop(0, GCH, _zr, 0)
        pltpu.sync_copy(rows_v, h0_out.at[pl.ds(N_NODES + wid * GCH, GCH), :])


@functools.partial(
    pl.kernel,
    out_type=(jax.ShapeDtypeStruct((NP, D), jnp.float32),
              jax.ShapeDtypeStruct((NP, D), jnp.float32)),
    mesh=_mesh(),
    scratch_types=[
        pltpu.VMEM((ECH,), jnp.int32),
        pltpu.VMEM((ECH,), jnp.int32),
        pltpu.VMEM((ECH, D), jnp.float32),
        pltpu.VMEM_SHARED((NP, D), jnp.float32),
        pltpu.SemaphoreType.DMA,
    ],
)
def _sc_edge_pass(u, src, dst, va_out, vb_out, sidx_v, didx_v, msg_v,
                  agg_sh, sem):
    c = lax.axis_index("c")
    s = lax.axis_index("s")

    r0 = s * RPT
    pltpu.sync_copy(u.at[pl.ds(r0, RPT)], agg_sh.at[pl.ds(r0, RPT)])
    plsc.subcore_barrier()

    ebase = c * (N_EDGES // SC_CORES) + s * EDGE_EPT

    def _edge(i, _):
        e0 = ebase + i * ECH
        pltpu.sync_copy(src.at[pl.ds(e0, ECH)], sidx_v)
        pltpu.sync_copy(dst.at[pl.ds(e0, ECH)], didx_v)
        pltpu.async_copy(u.at[sidx_v], msg_v, sem).wait()
        pltpu.sync_copy(msg_v, agg_sh.at[didx_v], add=True)
        return 0
    lax.fori_loop(0, EDGE_NCH, _edge, 0)
    plsc.subcore_barrier()

    @pl.when(c == 0)
    def _():
        pltpu.sync_copy(agg_sh.at[pl.ds(r0, RPT)], va_out.at[pl.ds(r0, RPT)])

    @pl.when(c == 1)
    def _():
        pltpu.sync_copy(agg_sh.at[pl.ds(r0, RPT)], vb_out.at[pl.ds(r0, RPT)])


@functools.partial(
    pl.kernel,
    out_type=jax.ShapeDtypeStruct((B, D), jnp.float32),
    mesh=_mesh(),
    scratch_types=[
        pltpu.VMEM((B // NW,), jnp.int32),
        pltpu.VMEM((B // NW, D), jnp.float32),
        pltpu.SemaphoreType.DMA,
    ],
)
def _sc_gather_rows(h2, outnodes, rep_out, idx_v, rows_v, sem):
    c = lax.axis_index("c")
    s = lax.axis_index("s")
    wid = c * SC_TILES + s
    n = B // NW
    base = wid * n
    pltpu.sync_copy(outnodes.at[pl.ds(base, n)], idx_v)
    pltpu.async_copy(h2.at[idx_v], rows_v, sem).wait()
    pltpu.sync_copy(rows_v, rep_out.at[pl.ds(base, n), :])


def _prep_body(dega_ref, degb_ref, h0_ref, dis_ref, u0_ref):
    dis = lax.rsqrt(dega_ref[...] + degb_ref[...] + 1.0)
    dis_ref[...] = dis
    u0_ref[...] = h0_ref[...] * dis


def _tc_prep(dega_col, degb_col, h0):
    return pl.pallas_call(
        _prep_body,
        grid=(NP // RB,),
        in_specs=[
            pl.BlockSpec((RB, 1), lambda i: (i, 0)),
            pl.BlockSpec((RB, 1), lambda i: (i, 0)),
            pl.BlockSpec((RB, D), lambda i: (i, 0)),
        ],
        out_specs=[
            pl.BlockSpec((RB, 1), lambda i: (i, 0)),
            pl.BlockSpec((RB, D), lambda i: (i, 0)),
        ],
        out_shape=[
            jax.ShapeDtypeStruct((NP, 1), jnp.float32),
            jax.ShapeDtypeStruct((NP, D), jnp.float32),
        ],
    )(dega_col, degb_col, h0)


def _layer_body(va_ref, vb_ref, u_ref, dis_ref, w_ref, o_ref, *, scale_out):
    x = (va_ref[...] + vb_ref[...] - u_ref[...]) * dis_ref[...]
    y = jnp.maximum(
        jnp.dot(x, w_ref[...], preferred_element_type=jnp.float32), 0.0)
    if scale_out:
        y = y * dis_ref[...]
    o_ref[...] = y


def _tc_layer(va, vb, u, dis_col, w, scale_out):
    return pl.pallas_call(
        functools.partial(_layer_body, scale_out=scale_out),
        grid=(NP // RB,),
        in_specs=[
            pl.BlockSpec((RB, D), lambda i: (i, 0)),
            pl.BlockSpec((RB, D), lambda i: (i, 0)),
            pl.BlockSpec((RB, D), lambda i: (i, 0)),
            pl.BlockSpec((RB, 1), lambda i: (i, 0)),
            pl.BlockSpec((D, D), lambda i: (0, 0)),
        ],
        out_specs=pl.BlockSpec((RB, D), lambda i: (i, 0)),
        out_shape=jax.ShapeDtypeStruct((NP, D), jnp.float32),
    )(va, vb, u, dis_col, w)


def _loss_body(rep_ref, se_ref, soft_ref, hard_ref, wc1_ref, wc2_ref, bc_ref,
               o_ref):
    logits = (
        jnp.dot(rep_ref[...], wc1_ref[...], preferred_element_type=jnp.float32)
        + jnp.dot(se_ref[...], wc2_ref[...], preferred_element_type=jnp.float32)
        + bc_ref[...])
    m = jnp.max(logits, axis=-1, keepdims=True)
    lse = m + jnp.log(jnp.sum(jnp.exp(logits - m), axis=-1, keepdims=True))
    logp = logits - lse

    soft = soft_ref[...]
    sm = jnp.max(soft, axis=-1, keepdims=True)
    te = jnp.exp(soft - sm)
    t = te / jnp.sum(te, axis=-1, keepdims=True)
    kl = jnp.sum(t * (jnp.log(t + 1e-9) - logp)) / B

    cls = lax.broadcasted_iota(jnp.int32, (B, NCLS), 1)
    onehot = cls == hard_ref[...]
    ce = -jnp.sum(jnp.where(onehot, logp, 0.0)) / B
    o_ref[...] = jnp.reshape(kl + LAM * ce, (1, 1))


def _tc_loss(rep, se, soft, hard2d, wc1, wc2, bc2d):
    return pl.pallas_call(
        _loss_body,
        out_shape=jax.ShapeDtypeStruct((1, 1), jnp.float32),
    )(rep, se, soft, hard2d, wc1, wc2, bc2d)


def kernel(input_nodes, edge_index, output_nodes, sentence_emb, soft_labels,
           hard_labels, emb_table, W1, W2, Wc, bc):
    src = edge_index[0]
    dst = edge_index[1]

    h0, dega, degb = _sc_embed_deg(emb_table, input_nodes, dst)
    dis_col, u0 = _tc_prep(dega.reshape(NP, 1), degb.reshape(NP, 1), h0)

    va1, vb1 = _sc_edge_pass(u0, src, dst)
    u1 = _tc_layer(va1, vb1, u0, dis_col, W1, scale_out=True)

    va2, vb2 = _sc_edge_pass(u1, src, dst)
    h2 = _tc_layer(va2, vb2, u1, dis_col, W2, scale_out=False)

    rep = _sc_gather_rows(h2, output_nodes)

    loss = _tc_loss(rep, sentence_emb, soft_labels, hard_labels[:, None],
                    Wc[:D], Wc[D:], bc[None, :])
    return loss[0, 0]
